# Initial kernel scaffold; baseline (speedup 1.0000x reference)
import jax
import jax.numpy as jnp
from jax import lax
from jax.experimental import pallas as pl
from jax.experimental.pallas import tpu as pltpu
from jax.experimental.pallas import tpu_sc as plsc

N = 10000
E = 320000
NFEAT = 128
D = 8
NCLASS = 64

NC = 2
NS = 16
NW = NC * NS
EPW = E // NW
CH = 80
NCHUNK = EPW // CH
RPS = N // NS

_mesh = plsc.VectorSubcoreMesh(core_axis_name="c", subcore_axis_name="s")


def _spmm_body(table_hbm, src_hbm, dst_hbm, w_hbm, zeros_hbm, out_hbm,
               src_v, dst_v, w_v, rows_v, scaled_v, acc):
    c = lax.axis_index("c")
    s = lax.axis_index("s")
    wid = s * NC + c

    pltpu.sync_copy(src_hbm.at[wid], src_v)
    pltpu.sync_copy(dst_hbm.at[wid], dst_v)
    pltpu.sync_copy(w_hbm.at[wid], w_v)

    pltpu.sync_copy(zeros_hbm.at[pl.ds(s * RPS, RPS)],
                    acc.at[pl.ds(s * RPS, RPS)])
    plsc.subcore_barrier()

    iota = lax.iota(jnp.int32, 16)
    pat_row = iota >> 3
    pat_col = iota & 7

    def body(j, carry):
        pltpu.sync_copy(table_hbm.at[src_v.at[j]], rows_v)
        joff = j * CH
        for t in range(CH * D // 16):
            row_idx = pat_row + (2 * t)
            wvec = plsc.load_gather(w_v, [row_idx + joff])
            vals = plsc.load_gather(rows_v, [row_idx, pat_col])
            plsc.store_scatter(scaled_v, [row_idx, pat_col], vals * wvec)
        pltpu.sync_copy(scaled_v, acc.at[dst_v.at[j]], add=True)
        return carry

    lax.fori_loop(0, NCHUNK, body, 0)
    plsc.subcore_barrier()
    pltpu.sync_copy(acc.at[pl.ds(s * RPS, RPS)],
                    out_hbm.at[c, pl.ds(s * RPS, RPS)])


_spmm_sc = pl.kernel(
    _spmm_body,
    out_type=jax.ShapeDtypeStruct((NC, N, D), jnp.float32),
    mesh=_mesh,
    scratch_types=[
        pltpu.VMEM((NCHUNK, CH), jnp.int32),
        pltpu.VMEM((NCHUNK, CH), jnp.int32),
        pltpu.VMEM((EPW,), jnp.float32),
        pltpu.VMEM((CH, D), jnp.float32),
        pltpu.VMEM((CH, D), jnp.float32),
        pltpu.VMEM_SHARED((N, D), jnp.float32),
    ],
)


_RB = 1000
_NB = N // _RB


def _dense1_body(x_ref, w1_ref, wb_ref, s1_ref, xr_ref, st_ref):
    i = pl.program_id(0)
    xb = x_ref[...]
    wb = wb_ref[...]
    s1_ref[...] = jnp.dot(xb, w1_ref[...], preferred_element_type=jnp.float32)
    t = jnp.dot(xb, wb, preferred_element_type=jnp.float32)
    sos = jnp.dot(xb * xb, wb * wb, preferred_element_type=jnp.float32)
    xr = jnp.maximum(0.5 * (t * t - sos), 0.0)
    xr_ref[...] = xr
    ssum = jnp.broadcast_to(jnp.sum(xr, axis=0, keepdims=True), (8, D))
    ssq = jnp.broadcast_to(jnp.sum(xr * xr, axis=0, keepdims=True), (8, D))
    blk = jnp.concatenate([ssum, ssq], axis=0)

    @pl.when(i == 0)
    def _():
        st_ref[...] = jnp.zeros_like(st_ref)

    st_ref[...] += blk


def _dense1(x, W1, Wb):
    return pl.pallas_call(
        _dense1_body,
        grid=(_NB,),
        in_specs=[
            pl.BlockSpec((_RB, NFEAT), lambda i: (i, 0)),
            pl.BlockSpec((NFEAT, D), lambda i: (0, 0)),
            pl.BlockSpec((NFEAT, D), lambda i: (0, 0)),
        ],
        out_specs=[
            pl.BlockSpec((_RB, D), lambda i: (i, 0)),
            pl.BlockSpec((_RB, D), lambda i: (i, 0)),
            pl.BlockSpec((16, D), lambda i: (0, 0)),
        ],
        out_shape=[
            jax.ShapeDtypeStruct((N, D), jnp.float32),
            jax.ShapeDtypeStruct((N, D), jnp.float32),
            jax.ShapeDtypeStruct((16, D), jnp.float32),
        ],
    )(x, W1, Wb)


def _dense2_body(sp_ref, xr_ref, st_ref, b1_ref, g_ref, bt_ref, h_ref):
    p = sp_ref[0] + sp_ref[1]
    xl = jnp.maximum(p + b1_ref[0:1, :], 0.0)
    mean = st_ref[0:1, :] * (1.0 / N)
    var = st_ref[8:9, :] * (1.0 / N) - mean * mean
    inv = 1.0 / jnp.sqrt(var + 1e-5)
    xrn = g_ref[0:1, :] * (xr_ref[...] - mean) * inv + bt_ref[0:1, :]
    h_ref[...] = 0.5 * (xl + xrn)


def _dense2(sp, xr, st, b1b, gb, btb):
    return pl.pallas_call(
        _dense2_body,
        grid=(_NB,),
        in_specs=[
            pl.BlockSpec((NC, _RB, D), lambda i: (0, i, 0)),
            pl.BlockSpec((_RB, D), lambda i: (i, 0)),
            pl.BlockSpec((16, D), lambda i: (0, 0)),
            pl.BlockSpec((8, D), lambda i: (0, 0)),
            pl.BlockSpec((8, D), lambda i: (0, 0)),
            pl.BlockSpec((8, D), lambda i: (0, 0)),
        ],
        out_specs=pl.BlockSpec((_RB, D), lambda i: (i, 0)),
        out_shape=jax.ShapeDtypeStruct((N, D), jnp.float32),
    )(sp, xr, st, b1b, gb, btb)


def _dense3_body(sp_ref, w2_ref, b2_ref, o_ref):
    hsum = sp_ref[0] + sp_ref[1]
    logits = jnp.dot(hsum, w2_ref[...],
                     preferred_element_type=jnp.float32) + b2_ref[0:1, :]
    m = jnp.max(logits, axis=1, keepdims=True)
    shifted = logits - m
    lse = jnp.log(jnp.sum(jnp.exp(shifted), axis=1, keepdims=True))
    o_ref[...] = shifted - lse


def _dense3(sp, W2, b2b):
    return pl.pallas_call(
        _dense3_body,
        grid=(_NB,),
        in_specs=[
            pl.BlockSpec((NC, _RB, D), lambda i: (0, i, 0)),
            pl.BlockSpec((D, NCLASS), lambda i: (0, 0)),
            pl.BlockSpec((8, NCLASS), lambda i: (0, 0)),
        ],
        out_specs=pl.BlockSpec((_RB, NCLASS), lambda i: (i, 0)),
        out_shape=jax.ShapeDtypeStruct((N, NCLASS), jnp.float32),
    )(sp, W2, b2b)


def kernel(x, edge_index, edge_weight, W1, b1, W2, b2, Wb, gamma, beta):
    src = edge_index[0].reshape(NW, NCHUNK, CH)
    dst = edge_index[1].reshape(NW, NCHUNK, CH)
    w2d = edge_weight.reshape(NW, EPW)
    zeros = jnp.zeros((N, D), jnp.float32)
    b1b = jnp.broadcast_to(b1.reshape(1, D), (8, D))
    gb = jnp.broadcast_to(gamma.reshape(1, D), (8, D))
    btb = jnp.broadcast_to(beta.reshape(1, D), (8, D))
    b2b = jnp.broadcast_to(b2.reshape(1, NCLASS), (8, NCLASS))

    s1, xr, st = _dense1(x, W1, Wb)
    sp1 = _spmm_sc(s1, src, dst, w2d, zeros)
    h = _dense2(sp1, xr, st, b1b, gb, btb)
    sp2 = _spmm_sc(h, src, dst, w2d, zeros)
    return _dense3(sp2, W2, b2b)


# speedup vs baseline: 9.7895x; 9.7895x over previous
import functools

import jax
import jax.numpy as jnp
from jax import lax
from jax.experimental import pallas as pl
from jax.experimental.pallas import tpu as pltpu
from jax.experimental.pallas import tpu_sc as plsc

N = 10000
E = 320000
NFEAT = 128
D = 8
NCLASS = 64

NC = 2
NS = 16
NW = NC * NS
EPW = E // NW
CH = 80
NCHUNK = EPW // CH
RPS = 624
TAIL = N - NS * RPS

def _spmm_body(table_hbm, src_hbm, dst_hbm, w_hbm, zeros_hbm, out_hbm,
               src_v, dst_v, w_v, rows_v, scaled_v, acc):
    c = lax.axis_index("c")
    s = lax.axis_index("s")
    wid = s * NC + c

    pltpu.sync_copy(src_hbm.at[wid], src_v)
    pltpu.sync_copy(dst_hbm.at[wid], dst_v)
    pltpu.sync_copy(w_hbm.at[wid], w_v)

    pltpu.sync_copy(zeros_hbm.at[pl.ds(s * RPS, RPS)],
                    acc.at[pl.ds(s * RPS, RPS)])

    @pl.when(s == NS - 1)
    def _():
        pltpu.sync_copy(zeros_hbm.at[pl.ds(NS * RPS, TAIL)],
                        acc.at[pl.ds(NS * RPS, TAIL)])

    plsc.subcore_barrier()

    iota = lax.iota(jnp.int32, 16)
    pat_row = iota >> 3
    pat_col = iota & 7

    def body(j, carry):
        pltpu.sync_copy(table_hbm.at[src_v.at[j]], rows_v)
        joff = j * CH
        for t in range(CH * D // 16):
            row_idx = pat_row + (2 * t)
            wvec = plsc.load_gather(w_v, [row_idx + joff])
            vals = plsc.load_gather(rows_v, [row_idx, pat_col])
            plsc.store_scatter(scaled_v, [row_idx, pat_col], vals * wvec)
        pltpu.sync_copy(scaled_v, acc.at[dst_v.at[j]], add=True)
        return carry

    lax.fori_loop(0, NCHUNK, body, 0)
    plsc.subcore_barrier()
    pltpu.sync_copy(acc.at[pl.ds(s * RPS, RPS)],
                    out_hbm.at[c, pl.ds(s * RPS, RPS)])

    @pl.when(s == NS - 1)
    def _():
        pltpu.sync_copy(acc.at[pl.ds(NS * RPS, TAIL)],
                        out_hbm.at[c, pl.ds(NS * RPS, TAIL)])


@functools.cache
def _get_spmm_sc():
    mesh = plsc.VectorSubcoreMesh(core_axis_name="c", subcore_axis_name="s",
                                  num_cores=NC, num_subcores=NS)
    return pl.kernel(
        _spmm_body,
        out_type=jax.ShapeDtypeStruct((NC, N, D), jnp.float32),
        mesh=mesh,
        compiler_params=pltpu.CompilerParams(needs_layout_passes=False,
                                             use_tc_tiling_on_sc=False),
        scratch_types=[
            pltpu.VMEM((NCHUNK, CH), jnp.int32),
            pltpu.VMEM((NCHUNK, CH), jnp.int32),
            pltpu.VMEM((EPW,), jnp.float32),
            pltpu.VMEM((CH, D), jnp.float32),
            pltpu.VMEM((CH, D), jnp.float32),
            pltpu.VMEM_SHARED((N, D), jnp.float32),
        ],
    )


_RB = 1000
_NB = N // _RB


def _dense1_body(x_ref, w1_ref, wb_ref, s1_ref, xr_ref, st_ref):
    i = pl.program_id(0)
    xb = x_ref[...]
    wb = wb_ref[...]
    s1_ref[...] = jnp.dot(xb, w1_ref[...], preferred_element_type=jnp.float32)
    t = jnp.dot(xb, wb, preferred_element_type=jnp.float32)
    sos = jnp.dot(xb * xb, wb * wb, preferred_element_type=jnp.float32)
    xr = jnp.maximum(0.5 * (t * t - sos), 0.0)
    xr_ref[...] = xr
    ssum = jnp.broadcast_to(jnp.sum(xr, axis=0, keepdims=True), (8, D))
    ssq = jnp.broadcast_to(jnp.sum(xr * xr, axis=0, keepdims=True), (8, D))
    blk = jnp.concatenate([ssum, ssq], axis=0)

    @pl.when(i == 0)
    def _():
        st_ref[...] = jnp.zeros_like(st_ref)

    st_ref[...] += blk


def _dense1(x, W1, Wb):
    return pl.pallas_call(
        _dense1_body,
        grid=(_NB,),
        in_specs=[
            pl.BlockSpec((_RB, NFEAT), lambda i: (i, 0)),
            pl.BlockSpec((NFEAT, D), lambda i: (0, 0)),
            pl.BlockSpec((NFEAT, D), lambda i: (0, 0)),
        ],
        out_specs=[
            pl.BlockSpec((_RB, D), lambda i: (i, 0)),
            pl.BlockSpec((_RB, D), lambda i: (i, 0)),
            pl.BlockSpec((16, D), lambda i: (0, 0)),
        ],
        out_shape=[
            jax.ShapeDtypeStruct((N, D), jnp.float32),
            jax.ShapeDtypeStruct((N, D), jnp.float32),
            jax.ShapeDtypeStruct((16, D), jnp.float32),
        ],
    )(x, W1, Wb)


def _dense2_body(sp_ref, xr_ref, st_ref, b1_ref, g_ref, bt_ref, h_ref):
    p = sp_ref[0] + sp_ref[1]
    xl = jnp.maximum(p + b1_ref[0:1, :], 0.0)
    mean = st_ref[0:1, :] * (1.0 / N)
    var = st_ref[8:9, :] * (1.0 / N) - mean * mean
    inv = 1.0 / jnp.sqrt(var + 1e-5)
    xrn = g_ref[0:1, :] * (xr_ref[...] - mean) * inv + bt_ref[0:1, :]
    h_ref[...] = 0.5 * (xl + xrn)


def _dense2(sp, xr, st, b1b, gb, btb):
    return pl.pallas_call(
        _dense2_body,
        grid=(_NB,),
        in_specs=[
            pl.BlockSpec((NC, _RB, D), lambda i: (0, i, 0)),
            pl.BlockSpec((_RB, D), lambda i: (i, 0)),
            pl.BlockSpec((16, D), lambda i: (0, 0)),
            pl.BlockSpec((8, D), lambda i: (0, 0)),
            pl.BlockSpec((8, D), lambda i: (0, 0)),
            pl.BlockSpec((8, D), lambda i: (0, 0)),
        ],
        out_specs=pl.BlockSpec((_RB, D), lambda i: (i, 0)),
        out_shape=jax.ShapeDtypeStruct((N, D), jnp.float32),
    )(sp, xr, st, b1b, gb, btb)


def _dense3_body(sp_ref, w2_ref, b2_ref, o_ref):
    hsum = sp_ref[0] + sp_ref[1]
    logits = jnp.dot(hsum, w2_ref[...],
                     preferred_element_type=jnp.float32) + b2_ref[0:1, :]
    m = jnp.max(logits, axis=1, keepdims=True)
    shifted = logits - m
    lse = jnp.log(jnp.sum(jnp.exp(shifted), axis=1, keepdims=True))
    o_ref[...] = shifted - lse


def _dense3(sp, W2, b2b):
    return pl.pallas_call(
        _dense3_body,
        grid=(_NB,),
        in_specs=[
            pl.BlockSpec((NC, _RB, D), lambda i: (0, i, 0)),
            pl.BlockSpec((D, NCLASS), lambda i: (0, 0)),
            pl.BlockSpec((8, NCLASS), lambda i: (0, 0)),
        ],
        out_specs=pl.BlockSpec((_RB, NCLASS), lambda i: (i, 0)),
        out_shape=jax.ShapeDtypeStruct((N, NCLASS), jnp.float32),
    )(sp, W2, b2b)


def kernel(x, edge_index, edge_weight, W1, b1, W2, b2, Wb, gamma, beta):
    src = edge_index[0].reshape(NW, NCHUNK, CH)
    dst = edge_index[1].reshape(NW, NCHUNK, CH)
    w2d = edge_weight.reshape(NW, EPW)
    zeros = jnp.zeros((N, D), jnp.float32)
    b1b = jnp.broadcast_to(b1.reshape(1, D), (8, D))
    gb = jnp.broadcast_to(gamma.reshape(1, D), (8, D))
    btb = jnp.broadcast_to(beta.reshape(1, D), (8, D))
    b2b = jnp.broadcast_to(b2.reshape(1, NCLASS), (8, NCLASS))

    spmm = _get_spmm_sc()
    s1, xr, st = _dense1(x, W1, Wb)
    sp1 = spmm(s1, src, dst, w2d, zeros)
    h = _dense2(sp1, xr, st, b1b, gb, btb)
    sp2 = spmm(h, src, dst, w2d, zeros)
    return _dense3(sp2, W2, b2b)


# speedup vs baseline: 14.0460x; 1.4348x over previous
import functools

import jax
import jax.numpy as jnp
from jax import lax
from jax.experimental import pallas as pl
from jax.experimental.pallas import tpu as pltpu
from jax.experimental.pallas import tpu_sc as plsc

N = 10000
E = 320000
NFEAT = 128
D = 8
NCLASS = 64

NC = 2
NS = 16
NW = NC * NS
EPW = E // NW
CH = 100
CHD = CH * D
NCHUNK = EPW // CH
RPS = 624
TAIL = N - NS * RPS

def _spmm_body(table_hbm, src_hbm, dst_hbm, wexp_hbm, zeros_hbm, out_hbm,
               src_v, dst_v, rows0, rows1, wx0, wx1, sc0, sc1, acc,
               g0, g1, w0, w1, s0, s1):
    c = lax.axis_index("c")
    s = lax.axis_index("s")
    wid = s * NC + c

    pltpu.sync_copy(src_hbm.at[wid], src_v)
    pltpu.sync_copy(dst_hbm.at[wid], dst_v)

    pltpu.sync_copy(zeros_hbm.at[pl.ds(s * RPS, RPS)],
                    acc.at[pl.ds(s * RPS, RPS)])

    @pl.when(s == NS - 1)
    def _():
        pltpu.sync_copy(zeros_hbm.at[pl.ds(NS * RPS, TAIL)],
                        acc.at[pl.ds(NS * RPS, TAIL)])

    plsc.subcore_barrier()

    iota = lax.iota(jnp.int32, 16)
    pat_row = iota >> 3
    pat_col = iota & 7

    bufs = ((rows0, wx0, sc0, g0, w0, s0), (rows1, wx1, sc1, g1, w1, s1))

    for b, (rows, wx, scb, gs, ws, ss) in enumerate(bufs):
        pltpu.async_copy(table_hbm.at[src_v.at[b]], rows, gs)
        pltpu.async_copy(wexp_hbm.at[wid, b], wx, ws)

    def body(k, carry):
        for b, (rows, wx, scb, gs, ws, ss) in enumerate(bufs):
            j = 2 * k + b
            pltpu.make_async_copy(table_hbm.at[src_v.at[j]], rows, gs).wait()
            pltpu.make_async_copy(wexp_hbm.at[wid, j], wx, ws).wait()

            @pl.when(k > 0)
            def _():
                pltpu.make_async_copy(scb, acc.at[dst_v.at[j]], ss).wait()

            for t in range(CHD // 16):
                row_idx = pat_row + (2 * t)
                vals = plsc.load_gather(rows, [row_idx, pat_col])
                plsc.store_scatter(scb, [row_idx, pat_col],
                                   vals * wx[pl.ds(16 * t, 16)])

            nj = lax.select(j + 2 >= NCHUNK, j + 2 - NCHUNK, j + 2)
            pltpu.async_copy(table_hbm.at[src_v.at[nj]], rows, gs)
            pltpu.async_copy(wexp_hbm.at[wid, nj], wx, ws)
            pltpu.async_copy(scb, acc.at[dst_v.at[j]], ss, add=True)
        return carry

    lax.fori_loop(0, NCHUNK // 2, body, 0)

    for b, (rows, wx, scb, gs, ws, ss) in enumerate(bufs):
        pltpu.make_async_copy(table_hbm.at[src_v.at[b]], rows, gs).wait()
        pltpu.make_async_copy(wexp_hbm.at[wid, b], wx, ws).wait()
        pltpu.make_async_copy(scb, acc.at[dst_v.at[b]], ss).wait()

    plsc.subcore_barrier()
    pltpu.sync_copy(acc.at[pl.ds(s * RPS, RPS)],
                    out_hbm.at[c, pl.ds(s * RPS, RPS)])

    @pl.when(s == NS - 1)
    def _():
        pltpu.sync_copy(acc.at[pl.ds(NS * RPS, TAIL)],
                        out_hbm.at[c, pl.ds(NS * RPS, TAIL)])


@functools.cache
def _get_spmm_sc():
    mesh = plsc.VectorSubcoreMesh(core_axis_name="c", subcore_axis_name="s",
                                  num_cores=NC, num_subcores=NS)
    return pl.kernel(
        _spmm_body,
        out_type=jax.ShapeDtypeStruct((NC, N, D), jnp.float32),
        mesh=mesh,
        compiler_params=pltpu.CompilerParams(needs_layout_passes=False,
                                             use_tc_tiling_on_sc=False),
        scratch_types=[
            pltpu.VMEM((NCHUNK, CH), jnp.int32),
            pltpu.VMEM((NCHUNK, CH), jnp.int32),
            pltpu.VMEM((CH, D), jnp.float32),
            pltpu.VMEM((CH, D), jnp.float32),
            pltpu.VMEM((CHD,), jnp.float32),
            pltpu.VMEM((CHD,), jnp.float32),
            pltpu.VMEM((CH, D), jnp.float32),
            pltpu.VMEM((CH, D), jnp.float32),
            pltpu.VMEM_SHARED((N, D), jnp.float32),
            pltpu.SemaphoreType.DMA,
            pltpu.SemaphoreType.DMA,
            pltpu.SemaphoreType.DMA,
            pltpu.SemaphoreType.DMA,
            pltpu.SemaphoreType.DMA,
            pltpu.SemaphoreType.DMA,
        ],
    )


_RB = 1000
_NB = N // _RB


def _dense1_body(x_ref, w1_ref, wb_ref, s1_ref, xr_ref, st_ref):
    i = pl.program_id(0)
    xb = x_ref[...]
    wb = wb_ref[...]
    s1_ref[...] = jnp.dot(xb, w1_ref[...], preferred_element_type=jnp.float32)
    t = jnp.dot(xb, wb, preferred_element_type=jnp.float32)
    sos = jnp.dot(xb * xb, wb * wb, preferred_element_type=jnp.float32)
    xr = jnp.maximum(0.5 * (t * t - sos), 0.0)
    xr_ref[...] = xr
    ssum = jnp.broadcast_to(jnp.sum(xr, axis=0, keepdims=True), (8, D))
    ssq = jnp.broadcast_to(jnp.sum(xr * xr, axis=0, keepdims=True), (8, D))
    blk = jnp.concatenate([ssum, ssq], axis=0)

    @pl.when(i == 0)
    def _():
        st_ref[...] = jnp.zeros_like(st_ref)

    st_ref[...] += blk


def _dense1(x, W1, Wb):
    return pl.pallas_call(
        _dense1_body,
        grid=(_NB,),
        in_specs=[
            pl.BlockSpec((_RB, NFEAT), lambda i: (i, 0)),
            pl.BlockSpec((NFEAT, D), lambda i: (0, 0)),
            pl.BlockSpec((NFEAT, D), lambda i: (0, 0)),
        ],
        out_specs=[
            pl.BlockSpec((_RB, D), lambda i: (i, 0)),
            pl.BlockSpec((_RB, D), lambda i: (i, 0)),
            pl.BlockSpec((16, D), lambda i: (0, 0)),
        ],
        out_shape=[
            jax.ShapeDtypeStruct((N, D), jnp.float32),
            jax.ShapeDtypeStruct((N, D), jnp.float32),
            jax.ShapeDtypeStruct((16, D), jnp.float32),
        ],
    )(x, W1, Wb)


def _dense2_body(sp_ref, xr_ref, st_ref, b1_ref, g_ref, bt_ref, h_ref):
    p = sp_ref[0] + sp_ref[1]
    xl = jnp.maximum(p + b1_ref[0:1, :], 0.0)
    mean = st_ref[0:1, :] * (1.0 / N)
    var = st_ref[8:9, :] * (1.0 / N) - mean * mean
    inv = 1.0 / jnp.sqrt(var + 1e-5)
    xrn = g_ref[0:1, :] * (xr_ref[...] - mean) * inv + bt_ref[0:1, :]
    h_ref[...] = 0.5 * (xl + xrn)


def _dense2(sp, xr, st, b1b, gb, btb):
    return pl.pallas_call(
        _dense2_body,
        grid=(_NB,),
        in_specs=[
            pl.BlockSpec((NC, _RB, D), lambda i: (0, i, 0)),
            pl.BlockSpec((_RB, D), lambda i: (i, 0)),
            pl.BlockSpec((16, D), lambda i: (0, 0)),
            pl.BlockSpec((8, D), lambda i: (0, 0)),
            pl.BlockSpec((8, D), lambda i: (0, 0)),
            pl.BlockSpec((8, D), lambda i: (0, 0)),
        ],
        out_specs=pl.BlockSpec((_RB, D), lambda i: (i, 0)),
        out_shape=jax.ShapeDtypeStruct((N, D), jnp.float32),
    )(sp, xr, st, b1b, gb, btb)


def _dense3_body(sp_ref, w2_ref, b2_ref, o_ref):
    hsum = sp_ref[0] + sp_ref[1]
    logits = jnp.dot(hsum, w2_ref[...],
                     preferred_element_type=jnp.float32) + b2_ref[0:1, :]
    m = jnp.max(logits, axis=1, keepdims=True)
    shifted = logits - m
    lse = jnp.log(jnp.sum(jnp.exp(shifted), axis=1, keepdims=True))
    o_ref[...] = shifted - lse


def _dense3(sp, W2, b2b):
    return pl.pallas_call(
        _dense3_body,
        grid=(_NB,),
        in_specs=[
            pl.BlockSpec((NC, _RB, D), lambda i: (0, i, 0)),
            pl.BlockSpec((D, NCLASS), lambda i: (0, 0)),
            pl.BlockSpec((8, NCLASS), lambda i: (0, 0)),
        ],
        out_specs=pl.BlockSpec((_RB, NCLASS), lambda i: (i, 0)),
        out_shape=jax.ShapeDtypeStruct((N, NCLASS), jnp.float32),
    )(sp, W2, b2b)


def kernel(x, edge_index, edge_weight, W1, b1, W2, b2, Wb, gamma, beta):
    src = edge_index[0].reshape(NW, NCHUNK, CH)
    dst = edge_index[1].reshape(NW, NCHUNK, CH)
    wexp = jnp.repeat(edge_weight, D).reshape(NW, NCHUNK, CHD)
    zeros = jnp.zeros((N, D), jnp.float32)
    b1b = jnp.broadcast_to(b1.reshape(1, D), (8, D))
    gb = jnp.broadcast_to(gamma.reshape(1, D), (8, D))
    btb = jnp.broadcast_to(beta.reshape(1, D), (8, D))
    b2b = jnp.broadcast_to(b2.reshape(1, NCLASS), (8, NCLASS))

    spmm = _get_spmm_sc()
    s1, xr, st = _dense1(x, W1, Wb)
    sp1 = spmm(s1, src, dst, wexp, zeros)
    h = _dense2(sp1, xr, st, b1b, gb, btb)
    sp2 = spmm(h, src, dst, wexp, zeros)
    return _dense3(sp2, W2, b2b)


# speedup vs baseline: 14.9644x; 1.0654x over previous
import functools

import jax
import jax.numpy as jnp
from jax import lax
from jax.experimental import pallas as pl
from jax.experimental.pallas import tpu as pltpu
from jax.experimental.pallas import tpu_sc as plsc

N = 10000
E = 320000
NFEAT = 128
D = 8
NCLASS = 64

NC = 2
NS = 16
NW = NC * NS
EPW = E // NW
CH = 100
CHD = CH * D
NCHUNK = EPW // CH
RPS = 624
TAIL = N - NS * RPS

def _spmm_body(table_hbm, src_hbm, dst_hbm, w_hbm, zeros_hbm, out_hbm,
               src_v, dst_v, w_v, rows0, rows1, sc0, sc1, acc,
               g0, g1, s0, s1):
    c = lax.axis_index("c")
    s = lax.axis_index("s")
    wid = s * NC + c

    pltpu.sync_copy(src_hbm.at[wid], src_v)
    pltpu.sync_copy(dst_hbm.at[wid], dst_v)
    pltpu.sync_copy(w_hbm.at[wid], w_v)

    pltpu.sync_copy(zeros_hbm.at[pl.ds(s * RPS, RPS)],
                    acc.at[pl.ds(s * RPS, RPS)])

    @pl.when(s == NS - 1)
    def _():
        pltpu.sync_copy(zeros_hbm.at[pl.ds(NS * RPS, TAIL)],
                        acc.at[pl.ds(NS * RPS, TAIL)])

    plsc.subcore_barrier()

    iota = lax.iota(jnp.int32, 16)
    pat_row = iota >> 3
    pat_col = iota & 7

    bufs = ((rows0, sc0, g0, s0), (rows1, sc1, g1, s1))

    for b, (rows, scb, gs, ss) in enumerate(bufs):
        pltpu.async_copy(table_hbm.at[src_v.at[b]], rows, gs)

    def body(k, carry):
        for b, (rows, scb, gs, ss) in enumerate(bufs):
            j = 2 * k + b
            pltpu.make_async_copy(table_hbm.at[src_v.at[j]], rows, gs).wait()

            @pl.when(k > 0)
            def _():
                pltpu.make_async_copy(scb, acc.at[dst_v.at[j]], ss).wait()

            joff = j * CH
            for t in range(CHD // 16):
                row_idx = pat_row + (2 * t)
                wvec = plsc.load_gather(w_v, [row_idx + joff])
                vals = plsc.load_gather(rows, [row_idx, pat_col])
                plsc.store_scatter(scb, [row_idx, pat_col], vals * wvec)

            nj = lax.select(j + 2 >= NCHUNK, j + 2 - NCHUNK, j + 2)
            pltpu.async_copy(table_hbm.at[src_v.at[nj]], rows, gs)
            pltpu.async_copy(scb, acc.at[dst_v.at[j]], ss, add=True)
        return carry

    lax.fori_loop(0, NCHUNK // 2, body, 0)

    for b, (rows, scb, gs, ss) in enumerate(bufs):
        pltpu.make_async_copy(table_hbm.at[src_v.at[b]], rows, gs).wait()
        pltpu.make_async_copy(scb, acc.at[dst_v.at[b]], ss).wait()

    plsc.subcore_barrier()
    pltpu.sync_copy(acc.at[pl.ds(s * RPS, RPS)],
                    out_hbm.at[c, pl.ds(s * RPS, RPS)])

    @pl.when(s == NS - 1)
    def _():
        pltpu.sync_copy(acc.at[pl.ds(NS * RPS, TAIL)],
                        out_hbm.at[c, pl.ds(NS * RPS, TAIL)])


@functools.cache
def _get_spmm_sc():
    mesh = plsc.VectorSubcoreMesh(core_axis_name="c", subcore_axis_name="s",
                                  num_cores=NC, num_subcores=NS)
    return pl.kernel(
        _spmm_body,
        out_type=jax.ShapeDtypeStruct((NC, N, D), jnp.float32),
        mesh=mesh,
        compiler_params=pltpu.CompilerParams(needs_layout_passes=False,
                                             use_tc_tiling_on_sc=False),
        scratch_types=[
            pltpu.VMEM((NCHUNK, CH), jnp.int32),
            pltpu.VMEM((NCHUNK, CH), jnp.int32),
            pltpu.VMEM((EPW,), jnp.float32),
            pltpu.VMEM((CH, D), jnp.float32),
            pltpu.VMEM((CH, D), jnp.float32),
            pltpu.VMEM((CH, D), jnp.float32),
            pltpu.VMEM((CH, D), jnp.float32),
            pltpu.VMEM_SHARED((N, D), jnp.float32),
            pltpu.SemaphoreType.DMA,
            pltpu.SemaphoreType.DMA,
            pltpu.SemaphoreType.DMA,
            pltpu.SemaphoreType.DMA,
        ],
    )


_RB = 1000
_NB = N // _RB


def _dense1_body(x_ref, w1_ref, wb_ref, s1_ref, xr_ref, st_ref):
    i = pl.program_id(0)
    xb = x_ref[...]
    wb = wb_ref[...]
    s1_ref[...] = jnp.dot(xb, w1_ref[...], preferred_element_type=jnp.float32)
    t = jnp.dot(xb, wb, preferred_element_type=jnp.float32)
    sos = jnp.dot(xb * xb, wb * wb, preferred_element_type=jnp.float32)
    xr = jnp.maximum(0.5 * (t * t - sos), 0.0)
    xr_ref[...] = xr
    ssum = jnp.broadcast_to(jnp.sum(xr, axis=0, keepdims=True), (8, D))
    ssq = jnp.broadcast_to(jnp.sum(xr * xr, axis=0, keepdims=True), (8, D))
    blk = jnp.concatenate([ssum, ssq], axis=0)

    @pl.when(i == 0)
    def _():
        st_ref[...] = jnp.zeros_like(st_ref)

    st_ref[...] += blk


def _dense1(x, W1, Wb):
    return pl.pallas_call(
        _dense1_body,
        grid=(_NB,),
        in_specs=[
            pl.BlockSpec((_RB, NFEAT), lambda i: (i, 0)),
            pl.BlockSpec((NFEAT, D), lambda i: (0, 0)),
            pl.BlockSpec((NFEAT, D), lambda i: (0, 0)),
        ],
        out_specs=[
            pl.BlockSpec((_RB, D), lambda i: (i, 0)),
            pl.BlockSpec((_RB, D), lambda i: (i, 0)),
            pl.BlockSpec((16, D), lambda i: (0, 0)),
        ],
        out_shape=[
            jax.ShapeDtypeStruct((N, D), jnp.float32),
            jax.ShapeDtypeStruct((N, D), jnp.float32),
            jax.ShapeDtypeStruct((16, D), jnp.float32),
        ],
    )(x, W1, Wb)


def _dense2_body(sp_ref, xr_ref, st_ref, b1_ref, g_ref, bt_ref, h_ref):
    p = sp_ref[0] + sp_ref[1]
    xl = jnp.maximum(p + b1_ref[0:1, :], 0.0)
    mean = st_ref[0:1, :] * (1.0 / N)
    var = st_ref[8:9, :] * (1.0 / N) - mean * mean
    inv = 1.0 / jnp.sqrt(var + 1e-5)
    xrn = g_ref[0:1, :] * (xr_ref[...] - mean) * inv + bt_ref[0:1, :]
    h_ref[...] = 0.5 * (xl + xrn)


def _dense2(sp, xr, st, b1b, gb, btb):
    return pl.pallas_call(
        _dense2_body,
        grid=(_NB,),
        in_specs=[
            pl.BlockSpec((NC, _RB, D), lambda i: (0, i, 0)),
            pl.BlockSpec((_RB, D), lambda i: (i, 0)),
            pl.BlockSpec((16, D), lambda i: (0, 0)),
            pl.BlockSpec((8, D), lambda i: (0, 0)),
            pl.BlockSpec((8, D), lambda i: (0, 0)),
            pl.BlockSpec((8, D), lambda i: (0, 0)),
        ],
        out_specs=pl.BlockSpec((_RB, D), lambda i: (i, 0)),
        out_shape=jax.ShapeDtypeStruct((N, D), jnp.float32),
    )(sp, xr, st, b1b, gb, btb)


def _dense3_body(sp_ref, w2_ref, b2_ref, o_ref):
    hsum = sp_ref[0] + sp_ref[1]
    logits = jnp.dot(hsum, w2_ref[...],
                     preferred_element_type=jnp.float32) + b2_ref[0:1, :]
    m = jnp.max(logits, axis=1, keepdims=True)
    shifted = logits - m
    lse = jnp.log(jnp.sum(jnp.exp(shifted), axis=1, keepdims=True))
    o_ref[...] = shifted - lse


def _dense3(sp, W2, b2b):
    return pl.pallas_call(
        _dense3_body,
        grid=(_NB,),
        in_specs=[
            pl.BlockSpec((NC, _RB, D), lambda i: (0, i, 0)),
            pl.BlockSpec((D, NCLASS), lambda i: (0, 0)),
            pl.BlockSpec((8, NCLASS), lambda i: (0, 0)),
        ],
        out_specs=pl.BlockSpec((_RB, NCLASS), lambda i: (i, 0)),
        out_shape=jax.ShapeDtypeStruct((N, NCLASS), jnp.float32),
    )(sp, W2, b2b)


def kernel(x, edge_index, edge_weight, W1, b1, W2, b2, Wb, gamma, beta):
    src = edge_index[0].reshape(NW, NCHUNK, CH)
    dst = edge_index[1].reshape(NW, NCHUNK, CH)
    w2d = edge_weight.reshape(NW, EPW)
    zeros = jnp.zeros((N, D), jnp.float32)
    b1b = jnp.broadcast_to(b1.reshape(1, D), (8, D))
    gb = jnp.broadcast_to(gamma.reshape(1, D), (8, D))
    btb = jnp.broadcast_to(beta.reshape(1, D), (8, D))
    b2b = jnp.broadcast_to(b2.reshape(1, NCLASS), (8, NCLASS))

    spmm = _get_spmm_sc()
    s1, xr, st = _dense1(x, W1, Wb)
    sp1 = spmm(s1, src, dst, w2d, zeros)
    h = _dense2(sp1, xr, st, b1b, gb, btb)
    sp2 = spmm(h, src, dst, w2d, zeros)
    return _dense3(sp2, W2, b2b)


# speedup vs baseline: 15.5841x; 1.0414x over previous
import functools

import jax
import jax.numpy as jnp
from jax import lax
from jax.experimental import pallas as pl
from jax.experimental.pallas import tpu as pltpu
from jax.experimental.pallas import tpu_sc as plsc

N = 10000
E = 320000
NFEAT = 128
D = 8
NCLASS = 64

NC = 2
NS = 16
NW = NC * NS
EPW = E // NW
CH = 100
CHD = CH * D
NCHUNK = EPW // CH
RPS = 624
TAIL = N - NS * RPS

def _spmm_body(table_hbm, edge_hbm, w_hbm, zeros_hbm, out_hbm,
               src_v, dst_v, w_v, rows0, rows1, sc0, sc1, acc,
               g0, g1, s0, s1):
    c = lax.axis_index("c")
    s = lax.axis_index("s")
    wid = s * NC + c

    pltpu.sync_copy(edge_hbm.at[0, wid], src_v)
    pltpu.sync_copy(edge_hbm.at[1, wid], dst_v)
    pltpu.sync_copy(w_hbm.at[wid], w_v)

    pltpu.sync_copy(zeros_hbm.at[pl.ds(s * RPS, RPS)],
                    acc.at[pl.ds(s * RPS, RPS)])

    @pl.when(s == NS - 1)
    def _():
        pltpu.sync_copy(zeros_hbm.at[pl.ds(NS * RPS, TAIL)],
                        acc.at[pl.ds(NS * RPS, TAIL)])

    plsc.subcore_barrier()

    iota = lax.iota(jnp.int32, 16)
    pat_row = iota >> 3
    pat_col = iota & 7

    bufs = ((rows0, sc0, g0, s0), (rows1, sc1, g1, s1))

    for b, (rows, scb, gs, ss) in enumerate(bufs):
        pltpu.async_copy(table_hbm.at[src_v.at[b]], rows, gs)

    def body(k, carry):
        for b, (rows, scb, gs, ss) in enumerate(bufs):
            j = 2 * k + b
            pltpu.make_async_copy(table_hbm.at[src_v.at[j]], rows, gs).wait()

            @pl.when(k > 0)
            def _():
                pltpu.make_async_copy(scb, acc.at[dst_v.at[j]], ss).wait()

            joff = j * CH
            for t in range(CHD // 16):
                row_idx = pat_row + (2 * t)
                wvec = plsc.load_gather(w_v, [row_idx + joff])
                vals = plsc.load_gather(rows, [row_idx, pat_col])
                plsc.store_scatter(scb, [row_idx, pat_col], vals * wvec)

            nj = lax.select(j + 2 >= NCHUNK, j + 2 - NCHUNK, j + 2)
            pltpu.async_copy(table_hbm.at[src_v.at[nj]], rows, gs)
            pltpu.async_copy(scb, acc.at[dst_v.at[j]], ss, add=True)
        return carry

    lax.fori_loop(0, NCHUNK // 2, body, 0)

    for b, (rows, scb, gs, ss) in enumerate(bufs):
        pltpu.make_async_copy(table_hbm.at[src_v.at[b]], rows, gs).wait()
        pltpu.make_async_copy(scb, acc.at[dst_v.at[b]], ss).wait()

    plsc.subcore_barrier()
    pltpu.sync_copy(acc.at[pl.ds(s * RPS, RPS)],
                    out_hbm.at[c, pl.ds(s * RPS, RPS)])

    @pl.when(s == NS - 1)
    def _():
        pltpu.sync_copy(acc.at[pl.ds(NS * RPS, TAIL)],
                        out_hbm.at[c, pl.ds(NS * RPS, TAIL)])


@functools.cache
def _get_spmm_sc():
    mesh = plsc.VectorSubcoreMesh(core_axis_name="c", subcore_axis_name="s",
                                  num_cores=NC, num_subcores=NS)
    return pl.kernel(
        _spmm_body,
        out_type=jax.ShapeDtypeStruct((NC, N, D), jnp.float32),
        mesh=mesh,
        compiler_params=pltpu.CompilerParams(needs_layout_passes=False,
                                             use_tc_tiling_on_sc=False),
        scratch_types=[
            pltpu.VMEM((NCHUNK, CH), jnp.int32),
            pltpu.VMEM((NCHUNK, CH), jnp.int32),
            pltpu.VMEM((EPW,), jnp.float32),
            pltpu.VMEM((CH, D), jnp.float32),
            pltpu.VMEM((CH, D), jnp.float32),
            pltpu.VMEM((CH, D), jnp.float32),
            pltpu.VMEM((CH, D), jnp.float32),
            pltpu.VMEM_SHARED((N, D), jnp.float32),
            pltpu.SemaphoreType.DMA,
            pltpu.SemaphoreType.DMA,
            pltpu.SemaphoreType.DMA,
            pltpu.SemaphoreType.DMA,
        ],
    )


_RB = 1000
_NB = N // _RB


def _dense1_body(x_ref, w1_ref, wb_ref, s1_ref, xr_ref, st_ref):
    i = pl.program_id(0)
    xb = x_ref[...]
    wb = wb_ref[...]
    s1_ref[...] = jnp.dot(xb, w1_ref[...], preferred_element_type=jnp.float32)
    t = jnp.dot(xb, wb, preferred_element_type=jnp.float32)
    sos = jnp.dot(xb * xb, wb * wb, preferred_element_type=jnp.float32)
    xr = jnp.maximum(0.5 * (t * t - sos), 0.0)
    xr_ref[...] = xr
    ssum = jnp.broadcast_to(jnp.sum(xr, axis=0, keepdims=True), (8, D))
    ssq = jnp.broadcast_to(jnp.sum(xr * xr, axis=0, keepdims=True), (8, D))
    blk = jnp.concatenate([ssum, ssq], axis=0)

    @pl.when(i == 0)
    def _():
        st_ref[...] = jnp.zeros_like(st_ref)

    st_ref[...] += blk


def _dense1(x, W1, Wb):
    return pl.pallas_call(
        _dense1_body,
        grid=(_NB,),
        in_specs=[
            pl.BlockSpec((_RB, NFEAT), lambda i: (i, 0)),
            pl.BlockSpec((NFEAT, D), lambda i: (0, 0)),
            pl.BlockSpec((NFEAT, D), lambda i: (0, 0)),
        ],
        out_specs=[
            pl.BlockSpec((_RB, D), lambda i: (i, 0)),
            pl.BlockSpec((_RB, D), lambda i: (i, 0)),
            pl.BlockSpec((16, D), lambda i: (0, 0)),
        ],
        out_shape=[
            jax.ShapeDtypeStruct((N, D), jnp.float32),
            jax.ShapeDtypeStruct((N, D), jnp.float32),
            jax.ShapeDtypeStruct((16, D), jnp.float32),
        ],
    )(x, W1, Wb)


def _dense2_body(sp_ref, xr_ref, st_ref, b1_ref, g_ref, bt_ref, h_ref):
    p = sp_ref[0] + sp_ref[1]
    xl = jnp.maximum(p + b1_ref[0:1, :], 0.0)
    mean = st_ref[0:1, :] * (1.0 / N)
    var = st_ref[8:9, :] * (1.0 / N) - mean * mean
    inv = 1.0 / jnp.sqrt(var + 1e-5)
    xrn = g_ref[0:1, :] * (xr_ref[...] - mean) * inv + bt_ref[0:1, :]
    h_ref[...] = 0.5 * (xl + xrn)


def _dense2(sp, xr, st, b1b, gb, btb):
    return pl.pallas_call(
        _dense2_body,
        grid=(_NB,),
        in_specs=[
            pl.BlockSpec((NC, _RB, D), lambda i: (0, i, 0)),
            pl.BlockSpec((_RB, D), lambda i: (i, 0)),
            pl.BlockSpec((16, D), lambda i: (0, 0)),
            pl.BlockSpec((8, D), lambda i: (0, 0)),
            pl.BlockSpec((8, D), lambda i: (0, 0)),
            pl.BlockSpec((8, D), lambda i: (0, 0)),
        ],
        out_specs=pl.BlockSpec((_RB, D), lambda i: (i, 0)),
        out_shape=jax.ShapeDtypeStruct((N, D), jnp.float32),
    )(sp, xr, st, b1b, gb, btb)


def _dense3_body(sp_ref, w2_ref, b2_ref, o_ref):
    hsum = sp_ref[0] + sp_ref[1]
    logits = jnp.dot(hsum, w2_ref[...],
                     preferred_element_type=jnp.float32) + b2_ref[0:1, :]
    m = jnp.max(logits, axis=1, keepdims=True)
    shifted = logits - m
    lse = jnp.log(jnp.sum(jnp.exp(shifted), axis=1, keepdims=True))
    o_ref[...] = shifted - lse


def _dense3(sp, W2, b2b):
    return pl.pallas_call(
        _dense3_body,
        grid=(_NB,),
        in_specs=[
            pl.BlockSpec((NC, _RB, D), lambda i: (0, i, 0)),
            pl.BlockSpec((D, NCLASS), lambda i: (0, 0)),
            pl.BlockSpec((8, NCLASS), lambda i: (0, 0)),
        ],
        out_specs=pl.BlockSpec((_RB, NCLASS), lambda i: (i, 0)),
        out_shape=jax.ShapeDtypeStruct((N, NCLASS), jnp.float32),
    )(sp, W2, b2b)


def kernel(x, edge_index, edge_weight, W1, b1, W2, b2, Wb, gamma, beta):
    e4 = edge_index.reshape(2, NW, NCHUNK, CH)
    w2d = edge_weight.reshape(NW, EPW)
    zeros = jnp.zeros((N, D), jnp.float32)
    b1b = jnp.broadcast_to(b1.reshape(1, D), (8, D))
    gb = jnp.broadcast_to(gamma.reshape(1, D), (8, D))
    btb = jnp.broadcast_to(beta.reshape(1, D), (8, D))
    b2b = jnp.broadcast_to(b2.reshape(1, NCLASS), (8, NCLASS))

    spmm = _get_spmm_sc()
    s1, xr, st = _dense1(x, W1, Wb)
    sp1 = spmm(s1, e4, w2d, zeros)
    h = _dense2(sp1, xr, st, b1b, gb, btb)
    sp2 = spmm(h, e4, w2d, zeros)
    return _dense3(sp2, W2, b2b)


# speedup vs baseline: 18.9020x; 1.2129x over previous
import functools

import jax
import jax.numpy as jnp
from jax import lax
from jax.experimental import pallas as pl
from jax.experimental.pallas import tpu as pltpu
from jax.experimental.pallas import tpu_sc as plsc

N = 10000
E = 320000
NFEAT = 128
D = 8
NCLASS = 64

NC = 2
NS = 16
NW = NC * NS
EPW = E // NW
CH = 100
CHD = CH * D
NCHUNK = EPW // CH
RPS = 624
TAIL = N - NS * RPS

def _spmm_body(table_hbm, edge_hbm, w_hbm, zeros_hbm, out_hbm,
               src_v, dst_v, w_v, rows0, rows1, rows2, rows3,
               sc0, sc1, sc2, sc3, acc,
               g0, g1, g2, g3, s0, s1, s2, s3):
    c = lax.axis_index("c")
    s = lax.axis_index("s")
    wid = s * NC + c

    pltpu.sync_copy(edge_hbm.at[0, wid], src_v)
    pltpu.sync_copy(edge_hbm.at[1, wid], dst_v)
    pltpu.sync_copy(w_hbm.at[wid], w_v)

    pltpu.sync_copy(zeros_hbm.at[pl.ds(s * RPS, RPS)],
                    acc.at[pl.ds(s * RPS, RPS)])

    @pl.when(s == NS - 1)
    def _():
        pltpu.sync_copy(zeros_hbm.at[pl.ds(NS * RPS, TAIL)],
                        acc.at[pl.ds(NS * RPS, TAIL)])

    plsc.subcore_barrier()

    iota = lax.iota(jnp.int32, 16)
    pat_row = iota >> 3
    pat_col = iota & 7

    bufs = ((rows0, sc0, g0, s0), (rows1, sc1, g1, s1),
            (rows2, sc2, g2, s2), (rows3, sc3, g3, s3))
    nbuf = len(bufs)

    for b, (rows, scb, gs, ss) in enumerate(bufs):
        pltpu.async_copy(table_hbm.at[src_v.at[b]], rows, gs)

    def body(k, carry):
        for b, (rows, scb, gs, ss) in enumerate(bufs):
            j = nbuf * k + b
            pltpu.make_async_copy(table_hbm.at[src_v.at[j]], rows, gs).wait()

            @pl.when(k > 0)
            def _():
                pltpu.make_async_copy(scb, acc.at[dst_v.at[j]], ss).wait()

            joff = j * CH
            for t in range(CHD // 16):
                row_idx = pat_row + (2 * t)
                wvec = plsc.load_gather(w_v, [row_idx + joff])
                vals = plsc.load_gather(rows, [row_idx, pat_col])
                plsc.store_scatter(scb, [row_idx, pat_col], vals * wvec)

            nj = lax.select(j + nbuf >= NCHUNK, j + nbuf - NCHUNK, j + nbuf)
            pltpu.async_copy(table_hbm.at[src_v.at[nj]], rows, gs)
            pltpu.async_copy(scb, acc.at[dst_v.at[j]], ss, add=True)
        return carry

    lax.fori_loop(0, NCHUNK // nbuf, body, 0)

    for b, (rows, scb, gs, ss) in enumerate(bufs):
        pltpu.make_async_copy(table_hbm.at[src_v.at[b]], rows, gs).wait()
        pltpu.make_async_copy(scb, acc.at[dst_v.at[b]], ss).wait()

    plsc.subcore_barrier()
    pltpu.sync_copy(acc.at[pl.ds(s * RPS, RPS)],
                    out_hbm.at[c, pl.ds(s * RPS, RPS)])

    @pl.when(s == NS - 1)
    def _():
        pltpu.sync_copy(acc.at[pl.ds(NS * RPS, TAIL)],
                        out_hbm.at[c, pl.ds(NS * RPS, TAIL)])


@functools.cache
def _get_spmm_sc():
    mesh = plsc.VectorSubcoreMesh(core_axis_name="c", subcore_axis_name="s",
                                  num_cores=NC, num_subcores=NS)
    return pl.kernel(
        _spmm_body,
        out_type=jax.ShapeDtypeStruct((NC, N, D), jnp.float32),
        mesh=mesh,
        compiler_params=pltpu.CompilerParams(needs_layout_passes=False,
                                             use_tc_tiling_on_sc=False),
        scratch_types=[
            pltpu.VMEM((NCHUNK, CH), jnp.int32),
            pltpu.VMEM((NCHUNK, CH), jnp.int32),
            pltpu.VMEM((EPW,), jnp.float32),
            pltpu.VMEM((CH, D), jnp.float32),
            pltpu.VMEM((CH, D), jnp.float32),
            pltpu.VMEM((CH, D), jnp.float32),
            pltpu.VMEM((CH, D), jnp.float32),
            pltpu.VMEM((CH, D), jnp.float32),
            pltpu.VMEM((CH, D), jnp.float32),
            pltpu.VMEM((CH, D), jnp.float32),
            pltpu.VMEM((CH, D), jnp.float32),
            pltpu.VMEM_SHARED((N, D), jnp.float32),
            pltpu.SemaphoreType.DMA,
            pltpu.SemaphoreType.DMA,
            pltpu.SemaphoreType.DMA,
            pltpu.SemaphoreType.DMA,
            pltpu.SemaphoreType.DMA,
            pltpu.SemaphoreType.DMA,
            pltpu.SemaphoreType.DMA,
            pltpu.SemaphoreType.DMA,
        ],
    )


_RB = 1000
_NB = N // _RB


def _dense1_body(x_ref, w1_ref, wb_ref, s1_ref, xr_ref, st_ref):
    i = pl.program_id(0)
    xb = x_ref[...]
    wb = wb_ref[...]
    s1_ref[...] = jnp.dot(xb, w1_ref[...], preferred_element_type=jnp.float32)
    t = jnp.dot(xb, wb, preferred_element_type=jnp.float32)
    sos = jnp.dot(xb * xb, wb * wb, preferred_element_type=jnp.float32)
    xr = jnp.maximum(0.5 * (t * t - sos), 0.0)
    xr_ref[...] = xr
    ssum = jnp.broadcast_to(jnp.sum(xr, axis=0, keepdims=True), (8, D))
    ssq = jnp.broadcast_to(jnp.sum(xr * xr, axis=0, keepdims=True), (8, D))
    blk = jnp.concatenate([ssum, ssq], axis=0)

    @pl.when(i == 0)
    def _():
        st_ref[...] = jnp.zeros_like(st_ref)

    st_ref[...] += blk


def _dense1(x, W1, Wb):
    return pl.pallas_call(
        _dense1_body,
        grid=(_NB,),
        in_specs=[
            pl.BlockSpec((_RB, NFEAT), lambda i: (i, 0)),
            pl.BlockSpec((NFEAT, D), lambda i: (0, 0)),
            pl.BlockSpec((NFEAT, D), lambda i: (0, 0)),
        ],
        out_specs=[
            pl.BlockSpec((_RB, D), lambda i: (i, 0)),
            pl.BlockSpec((_RB, D), lambda i: (i, 0)),
            pl.BlockSpec((16, D), lambda i: (0, 0)),
        ],
        out_shape=[
            jax.ShapeDtypeStruct((N, D), jnp.float32),
            jax.ShapeDtypeStruct((N, D), jnp.float32),
            jax.ShapeDtypeStruct((16, D), jnp.float32),
        ],
    )(x, W1, Wb)


def _dense2_body(sp_ref, xr_ref, st_ref, b1_ref, g_ref, bt_ref, h_ref):
    p = sp_ref[0] + sp_ref[1]
    xl = jnp.maximum(p + b1_ref[0:1, :], 0.0)
    mean = st_ref[0:1, :] * (1.0 / N)
    var = st_ref[8:9, :] * (1.0 / N) - mean * mean
    inv = 1.0 / jnp.sqrt(var + 1e-5)
    xrn = g_ref[0:1, :] * (xr_ref[...] - mean) * inv + bt_ref[0:1, :]
    h_ref[...] = 0.5 * (xl + xrn)


def _dense2(sp, xr, st, b1b, gb, btb):
    return pl.pallas_call(
        _dense2_body,
        grid=(_NB,),
        in_specs=[
            pl.BlockSpec((NC, _RB, D), lambda i: (0, i, 0)),
            pl.BlockSpec((_RB, D), lambda i: (i, 0)),
            pl.BlockSpec((16, D), lambda i: (0, 0)),
            pl.BlockSpec((8, D), lambda i: (0, 0)),
            pl.BlockSpec((8, D), lambda i: (0, 0)),
            pl.BlockSpec((8, D), lambda i: (0, 0)),
        ],
        out_specs=pl.BlockSpec((_RB, D), lambda i: (i, 0)),
        out_shape=jax.ShapeDtypeStruct((N, D), jnp.float32),
    )(sp, xr, st, b1b, gb, btb)


def _dense3_body(sp_ref, w2_ref, b2_ref, o_ref):
    hsum = sp_ref[0] + sp_ref[1]
    logits = jnp.dot(hsum, w2_ref[...],
                     preferred_element_type=jnp.float32) + b2_ref[0:1, :]
    m = jnp.max(logits, axis=1, keepdims=True)
    shifted = logits - m
    lse = jnp.log(jnp.sum(jnp.exp(shifted), axis=1, keepdims=True))
    o_ref[...] = shifted - lse


def _dense3(sp, W2, b2b):
    return pl.pallas_call(
        _dense3_body,
        grid=(_NB,),
        in_specs=[
            pl.BlockSpec((NC, _RB, D), lambda i: (0, i, 0)),
            pl.BlockSpec((D, NCLASS), lambda i: (0, 0)),
            pl.BlockSpec((8, NCLASS), lambda i: (0, 0)),
        ],
        out_specs=pl.BlockSpec((_RB, NCLASS), lambda i: (i, 0)),
        out_shape=jax.ShapeDtypeStruct((N, NCLASS), jnp.float32),
    )(sp, W2, b2b)


def kernel(x, edge_index, edge_weight, W1, b1, W2, b2, Wb, gamma, beta):
    e4 = edge_index.reshape(2, NW, NCHUNK, CH)
    w2d = edge_weight.reshape(NW, EPW)
    zeros = jnp.zeros((N, D), jnp.float32)
    b1b = jnp.broadcast_to(b1.reshape(1, D), (8, D))
    gb = jnp.broadcast_to(gamma.reshape(1, D), (8, D))
    btb = jnp.broadcast_to(beta.reshape(1, D), (8, D))
    b2b = jnp.broadcast_to(b2.reshape(1, NCLASS), (8, NCLASS))

    spmm = _get_spmm_sc()
    s1, xr, st = _dense1(x, W1, Wb)
    sp1 = spmm(s1, e4, w2d, zeros)
    h = _dense2(sp1, xr, st, b1b, gb, btb)
    sp2 = spmm(h, e4, w2d, zeros)
    return _dense3(sp2, W2, b2b)


# speedup vs baseline: 19.2664x; 1.0193x over previous
import functools

import jax
import jax.numpy as jnp
from jax import lax
from jax.experimental import pallas as pl
from jax.experimental.pallas import tpu as pltpu
from jax.experimental.pallas import tpu_sc as plsc

N = 10000
E = 320000
NFEAT = 128
D = 8
NCLASS = 64

NC = 2
NS = 16
NW = NC * NS
EPW = E // NW
CH = 100
CHD = CH * D
NCHUNK = EPW // CH
RPS = 624
TAIL = N - NS * RPS

def _spmm_body(table_hbm, edge_hbm, w_hbm, zeros_hbm, out_hbm,
               src_v, dst_v, w_v, rows0, rows1, rows2, rows3, rows4,
               sc0, sc1, sc2, sc3, sc4, acc,
               g0, g1, g2, g3, g4, s0, s1, s2, s3, s4):
    c = lax.axis_index("c")
    s = lax.axis_index("s")
    wid = s * NC + c

    pltpu.sync_copy(edge_hbm.at[0, wid], src_v)
    pltpu.sync_copy(edge_hbm.at[1, wid], dst_v)
    pltpu.sync_copy(w_hbm.at[wid], w_v)

    pltpu.sync_copy(zeros_hbm.at[pl.ds(s * RPS, RPS)],
                    acc.at[pl.ds(s * RPS, RPS)])

    @pl.when(s == NS - 1)
    def _():
        pltpu.sync_copy(zeros_hbm.at[pl.ds(NS * RPS, TAIL)],
                        acc.at[pl.ds(NS * RPS, TAIL)])

    plsc.subcore_barrier()

    iota = lax.iota(jnp.int32, 16)
    pat_row = iota >> 3
    pat_col = iota & 7

    bufs = ((rows0, sc0, g0, s0), (rows1, sc1, g1, s1),
            (rows2, sc2, g2, s2), (rows3, sc3, g3, s3),
            (rows4, sc4, g4, s4))
    nbuf = len(bufs)

    for b, (rows, scb, gs, ss) in enumerate(bufs):
        pltpu.async_copy(table_hbm.at[src_v.at[b]], rows, gs)

    def body(k, carry):
        for b, (rows, scb, gs, ss) in enumerate(bufs):
            j = nbuf * k + b
            pltpu.make_async_copy(table_hbm.at[src_v.at[j]], rows, gs).wait()

            @pl.when(k > 0)
            def _():
                pltpu.make_async_copy(scb, acc.at[dst_v.at[j]], ss).wait()

            joff = j * CH
            for t in range(CHD // 16):
                row_idx = pat_row + (2 * t)
                wvec = plsc.load_gather(w_v, [row_idx + joff])
                vals = plsc.load_gather(rows, [row_idx, pat_col])
                plsc.store_scatter(scb, [row_idx, pat_col], vals * wvec)

            nj = lax.select(j + nbuf >= NCHUNK, j + nbuf - NCHUNK, j + nbuf)
            pltpu.async_copy(table_hbm.at[src_v.at[nj]], rows, gs)
            pltpu.async_copy(scb, acc.at[dst_v.at[j]], ss, add=True)
        return carry

    lax.fori_loop(0, NCHUNK // nbuf, body, 0)

    for b, (rows, scb, gs, ss) in enumerate(bufs):
        pltpu.make_async_copy(table_hbm.at[src_v.at[b]], rows, gs).wait()
        pltpu.make_async_copy(scb, acc.at[dst_v.at[b]], ss).wait()

    plsc.subcore_barrier()
    pltpu.sync_copy(acc.at[pl.ds(s * RPS, RPS)],
                    out_hbm.at[c, pl.ds(s * RPS, RPS)])

    @pl.when(s == NS - 1)
    def _():
        pltpu.sync_copy(acc.at[pl.ds(NS * RPS, TAIL)],
                        out_hbm.at[c, pl.ds(NS * RPS, TAIL)])


@functools.cache
def _get_spmm_sc():
    mesh = plsc.VectorSubcoreMesh(core_axis_name="c", subcore_axis_name="s",
                                  num_cores=NC, num_subcores=NS)
    return pl.kernel(
        _spmm_body,
        out_type=jax.ShapeDtypeStruct((NC, N, D), jnp.float32),
        mesh=mesh,
        compiler_params=pltpu.CompilerParams(needs_layout_passes=False,
                                             use_tc_tiling_on_sc=False),
        scratch_types=[
            pltpu.VMEM((NCHUNK, CH), jnp.int32),
            pltpu.VMEM((NCHUNK, CH), jnp.int32),
            pltpu.VMEM((EPW,), jnp.float32),
            pltpu.VMEM((CH, D), jnp.float32),
            pltpu.VMEM((CH, D), jnp.float32),
            pltpu.VMEM((CH, D), jnp.float32),
            pltpu.VMEM((CH, D), jnp.float32),
            pltpu.VMEM((CH, D), jnp.float32),
            pltpu.VMEM((CH, D), jnp.float32),
            pltpu.VMEM((CH, D), jnp.float32),
            pltpu.VMEM((CH, D), jnp.float32),
            pltpu.VMEM((CH, D), jnp.float32),
            pltpu.VMEM((CH, D), jnp.float32),
            pltpu.VMEM_SHARED((N, D), jnp.float32),
            pltpu.SemaphoreType.DMA,
            pltpu.SemaphoreType.DMA,
            pltpu.SemaphoreType.DMA,
            pltpu.SemaphoreType.DMA,
            pltpu.SemaphoreType.DMA,
            pltpu.SemaphoreType.DMA,
            pltpu.SemaphoreType.DMA,
            pltpu.SemaphoreType.DMA,
            pltpu.SemaphoreType.DMA,
            pltpu.SemaphoreType.DMA,
        ],
    )


_RB = 1000
_NB = N // _RB


def _dense1_body(x_ref, w1_ref, wb_ref, s1_ref, xr_ref, st_ref):
    i = pl.program_id(0)
    xb = x_ref[...]
    wb = wb_ref[...]
    s1_ref[...] = jnp.dot(xb, w1_ref[...], preferred_element_type=jnp.float32)
    t = jnp.dot(xb, wb, preferred_element_type=jnp.float32)
    sos = jnp.dot(xb * xb, wb * wb, preferred_element_type=jnp.float32)
    xr = jnp.maximum(0.5 * (t * t - sos), 0.0)
    xr_ref[...] = xr
    ssum = jnp.broadcast_to(jnp.sum(xr, axis=0, keepdims=True), (8, D))
    ssq = jnp.broadcast_to(jnp.sum(xr * xr, axis=0, keepdims=True), (8, D))
    blk = jnp.concatenate([ssum, ssq], axis=0)

    @pl.when(i == 0)
    def _():
        st_ref[...] = jnp.zeros_like(st_ref)

    st_ref[...] += blk


def _dense1(x, W1, Wb):
    return pl.pallas_call(
        _dense1_body,
        grid=(_NB,),
        in_specs=[
            pl.BlockSpec((_RB, NFEAT), lambda i: (i, 0)),
            pl.BlockSpec((NFEAT, D), lambda i: (0, 0)),
            pl.BlockSpec((NFEAT, D), lambda i: (0, 0)),
        ],
        out_specs=[
            pl.BlockSpec((_RB, D), lambda i: (i, 0)),
            pl.BlockSpec((_RB, D), lambda i: (i, 0)),
            pl.BlockSpec((16, D), lambda i: (0, 0)),
        ],
        out_shape=[
            jax.ShapeDtypeStruct((N, D), jnp.float32),
            jax.ShapeDtypeStruct((N, D), jnp.float32),
            jax.ShapeDtypeStruct((16, D), jnp.float32),
        ],
    )(x, W1, Wb)


def _dense2_body(sp_ref, xr_ref, st_ref, b1_ref, g_ref, bt_ref, h_ref):
    p = sp_ref[0] + sp_ref[1]
    xl = jnp.maximum(p + b1_ref[0:1, :], 0.0)
    mean = st_ref[0:1, :] * (1.0 / N)
    var = st_ref[8:9, :] * (1.0 / N) - mean * mean
    inv = 1.0 / jnp.sqrt(var + 1e-5)
    xrn = g_ref[0:1, :] * (xr_ref[...] - mean) * inv + bt_ref[0:1, :]
    h_ref[...] = 0.5 * (xl + xrn)


def _dense2(sp, xr, st, b1b, gb, btb):
    return pl.pallas_call(
        _dense2_body,
        grid=(_NB,),
        in_specs=[
            pl.BlockSpec((NC, _RB, D), lambda i: (0, i, 0)),
            pl.BlockSpec((_RB, D), lambda i: (i, 0)),
            pl.BlockSpec((16, D), lambda i: (0, 0)),
            pl.BlockSpec((8, D), lambda i: (0, 0)),
            pl.BlockSpec((8, D), lambda i: (0, 0)),
            pl.BlockSpec((8, D), lambda i: (0, 0)),
        ],
        out_specs=pl.BlockSpec((_RB, D), lambda i: (i, 0)),
        out_shape=jax.ShapeDtypeStruct((N, D), jnp.float32),
    )(sp, xr, st, b1b, gb, btb)


def _dense3_body(sp_ref, w2_ref, b2_ref, o_ref):
    hsum = sp_ref[0] + sp_ref[1]
    logits = jnp.dot(hsum, w2_ref[...],
                     preferred_element_type=jnp.float32) + b2_ref[0:1, :]
    m = jnp.max(logits, axis=1, keepdims=True)
    shifted = logits - m
    lse = jnp.log(jnp.sum(jnp.exp(shifted), axis=1, keepdims=True))
    o_ref[...] = shifted - lse


def _dense3(sp, W2, b2b):
    return pl.pallas_call(
        _dense3_body,
        grid=(_NB,),
        in_specs=[
            pl.BlockSpec((NC, _RB, D), lambda i: (0, i, 0)),
            pl.BlockSpec((D, NCLASS), lambda i: (0, 0)),
            pl.BlockSpec((8, NCLASS), lambda i: (0, 0)),
        ],
        out_specs=pl.BlockSpec((_RB, NCLASS), lambda i: (i, 0)),
        out_shape=jax.ShapeDtypeStruct((N, NCLASS), jnp.float32),
    )(sp, W2, b2b)


def kernel(x, edge_index, edge_weight, W1, b1, W2, b2, Wb, gamma, beta):
    e4 = edge_index.reshape(2, NW, NCHUNK, CH)
    w2d = edge_weight.reshape(NW, EPW)
    zeros = jnp.zeros((N, D), jnp.float32)
    b1b = jnp.broadcast_to(b1.reshape(1, D), (8, D))
    gb = jnp.broadcast_to(gamma.reshape(1, D), (8, D))
    btb = jnp.broadcast_to(beta.reshape(1, D), (8, D))
    b2b = jnp.broadcast_to(b2.reshape(1, NCLASS), (8, NCLASS))

    spmm = _get_spmm_sc()
    s1, xr, st = _dense1(x, W1, Wb)
    sp1 = spmm(s1, e4, w2d, zeros)
    h = _dense2(sp1, xr, st, b1b, gb, btb)
    sp2 = spmm(h, e4, w2d, zeros)
    return _dense3(sp2, W2, b2b)


# speedup vs baseline: 19.5510x; 1.0148x over previous
import functools

import jax
import jax.numpy as jnp
from jax import lax
from jax.experimental import pallas as pl
from jax.experimental.pallas import tpu as pltpu
from jax.experimental.pallas import tpu_sc as plsc

N = 10000
E = 320000
NFEAT = 128
D = 8
NCLASS = 64

NC = 2
NS = 16
NW = NC * NS
EPW = E // NW
CH = 100
CHD = CH * D
NCHUNK = EPW // CH
RPS = 624
TAIL = N - NS * RPS

def _spmm_body(table_hbm, edge_hbm, w_hbm, zeros_hbm, out_hbm,
               src_v, dst_v, w_v, rows0, rows1, rows2, rows3, rows4,
               sc0, sc1, sc2, sc3, sc4, acc,
               g0, g1, g2, g3, g4, s0, s1, s2, s3, s4):
    c = lax.axis_index("c")
    s = lax.axis_index("s")
    wid = s * NC + c

    pltpu.sync_copy(edge_hbm.at[0, wid], src_v)
    pltpu.sync_copy(edge_hbm.at[1, wid], dst_v)
    pltpu.sync_copy(w_hbm.at[wid], w_v)

    pltpu.sync_copy(zeros_hbm.at[pl.ds(s * RPS, RPS)],
                    acc.at[pl.ds(s * RPS, RPS)])

    @pl.when(s == NS - 1)
    def _():
        pltpu.sync_copy(zeros_hbm.at[pl.ds(NS * RPS, TAIL)],
                        acc.at[pl.ds(NS * RPS, TAIL)])

    plsc.subcore_barrier()

    iota = lax.iota(jnp.int32, 16)
    pat_row = iota >> 3
    pat_col = iota & 7

    bufs = ((rows0, sc0, g0, s0), (rows1, sc1, g1, s1),
            (rows2, sc2, g2, s2), (rows3, sc3, g3, s3),
            (rows4, sc4, g4, s4))
    nbuf = len(bufs)

    for b, (rows, scb, gs, ss) in enumerate(bufs):
        pltpu.async_copy(table_hbm.at[src_v.at[b]], rows, gs)

    def body(k, carry):
        for b, (rows, scb, gs, ss) in enumerate(bufs):
            j = nbuf * k + b
            pltpu.make_async_copy(table_hbm.at[src_v.at[j]], rows, gs).wait()

            @pl.when(k > 0)
            def _():
                pltpu.make_async_copy(scb, acc.at[dst_v.at[j]], ss).wait()

            joff = j * CH
            for t in range(CHD // 16):
                row_idx = pat_row + (2 * t)
                wvec = plsc.load_gather(w_v, [row_idx + joff])
                vals = plsc.load_gather(rows, [row_idx, pat_col])
                plsc.store_scatter(scb, [row_idx, pat_col], vals * wvec)

            nj = lax.select(j + nbuf >= NCHUNK, j + nbuf - NCHUNK, j + nbuf)
            pltpu.async_copy(table_hbm.at[src_v.at[nj]], rows, gs)
            pltpu.async_copy(scb, acc.at[dst_v.at[j]], ss, add=True)
        return carry

    lax.fori_loop(0, NCHUNK // nbuf, body, 0)

    for b, (rows, scb, gs, ss) in enumerate(bufs):
        pltpu.make_async_copy(table_hbm.at[src_v.at[b]], rows, gs).wait()
        pltpu.make_async_copy(scb, acc.at[dst_v.at[b]], ss).wait()

    plsc.subcore_barrier()
    pltpu.sync_copy(acc.at[pl.ds(s * RPS, RPS)],
                    out_hbm.at[c, pl.ds(s * RPS, RPS)])

    @pl.when(s == NS - 1)
    def _():
        pltpu.sync_copy(acc.at[pl.ds(NS * RPS, TAIL)],
                        out_hbm.at[c, pl.ds(NS * RPS, TAIL)])


@functools.cache
def _get_spmm_sc():
    mesh = plsc.VectorSubcoreMesh(core_axis_name="c", subcore_axis_name="s",
                                  num_cores=NC, num_subcores=NS)
    return pl.kernel(
        _spmm_body,
        out_type=jax.ShapeDtypeStruct((NC, N, D), jnp.float32),
        mesh=mesh,
        compiler_params=pltpu.CompilerParams(needs_layout_passes=False,
                                             use_tc_tiling_on_sc=False),
        scratch_types=[
            pltpu.VMEM((NCHUNK, CH), jnp.int32),
            pltpu.VMEM((NCHUNK, CH), jnp.int32),
            pltpu.VMEM((EPW,), jnp.float32),
            pltpu.VMEM((CH, D), jnp.float32),
            pltpu.VMEM((CH, D), jnp.float32),
            pltpu.VMEM((CH, D), jnp.float32),
            pltpu.VMEM((CH, D), jnp.float32),
            pltpu.VMEM((CH, D), jnp.float32),
            pltpu.VMEM((CH, D), jnp.float32),
            pltpu.VMEM((CH, D), jnp.float32),
            pltpu.VMEM((CH, D), jnp.float32),
            pltpu.VMEM((CH, D), jnp.float32),
            pltpu.VMEM((CH, D), jnp.float32),
            pltpu.VMEM_SHARED((N, D), jnp.float32),
            pltpu.SemaphoreType.DMA,
            pltpu.SemaphoreType.DMA,
            pltpu.SemaphoreType.DMA,
            pltpu.SemaphoreType.DMA,
            pltpu.SemaphoreType.DMA,
            pltpu.SemaphoreType.DMA,
            pltpu.SemaphoreType.DMA,
            pltpu.SemaphoreType.DMA,
            pltpu.SemaphoreType.DMA,
            pltpu.SemaphoreType.DMA,
        ],
    )


_RB = 1000
_NB = N // _RB


def _dense1_body(x_ref, w1_ref, wb_ref, s1_ref, xr_ref, st_ref):
    i = pl.program_id(0)
    xb = x_ref[...]
    wb = wb_ref[...]
    s1_ref[...] = jnp.dot(xb, w1_ref[...], preferred_element_type=jnp.float32)
    t = jnp.dot(xb, wb, preferred_element_type=jnp.float32)
    sos = jnp.dot(xb * xb, wb * wb, preferred_element_type=jnp.float32)
    xr = jnp.maximum(0.5 * (t * t - sos), 0.0)
    xr_ref[...] = xr
    ssum = jnp.broadcast_to(jnp.sum(xr, axis=0, keepdims=True), (8, D))
    ssq = jnp.broadcast_to(jnp.sum(xr * xr, axis=0, keepdims=True), (8, D))
    blk = jnp.concatenate([ssum, ssq], axis=0)

    @pl.when(i == 0)
    def _():
        st_ref[...] = jnp.zeros_like(st_ref)

    st_ref[...] += blk


def _dense1(x, W1, Wb):
    return pl.pallas_call(
        _dense1_body,
        grid=(_NB,),
        in_specs=[
            pl.BlockSpec((_RB, NFEAT), lambda i: (i, 0)),
            pl.BlockSpec((NFEAT, D), lambda i: (0, 0)),
            pl.BlockSpec((NFEAT, D), lambda i: (0, 0)),
        ],
        out_specs=[
            pl.BlockSpec((_RB, D), lambda i: (i, 0)),
            pl.BlockSpec((_RB, D), lambda i: (i, 0)),
            pl.BlockSpec((16, D), lambda i: (0, 0)),
        ],
        out_shape=[
            jax.ShapeDtypeStruct((N, D), jnp.float32),
            jax.ShapeDtypeStruct((N, D), jnp.float32),
            jax.ShapeDtypeStruct((16, D), jnp.float32),
        ],
    )(x, W1, Wb)


def _dense2_body(sp_ref, xr_ref, st_ref, b1_ref, g_ref, bt_ref, h_ref,
                 sp_v, h_v):
    pltpu.sync_copy(sp_ref, sp_v)
    p = sp_v[0] + sp_v[1]
    xl = jnp.maximum(p + b1_ref[0:1, :], 0.0)
    mean = st_ref[0:1, :] * (1.0 / N)
    var = st_ref[8:9, :] * (1.0 / N) - mean * mean
    inv = 1.0 / jnp.sqrt(var + 1e-5)
    xrn = g_ref[0:1, :] * (xr_ref[...] - mean) * inv + bt_ref[0:1, :]
    h_v[...] = 0.5 * (xl + xrn)
    pltpu.sync_copy(h_v, h_ref)


def _dense2(sp, xr, st, b1b, gb, btb):
    return pl.pallas_call(
        _dense2_body,
        grid=(1,),
        in_specs=[
            pl.BlockSpec(memory_space=pl.ANY),
            pl.BlockSpec((N, D), lambda i: (0, 0)),
            pl.BlockSpec((16, D), lambda i: (0, 0)),
            pl.BlockSpec((8, D), lambda i: (0, 0)),
            pl.BlockSpec((8, D), lambda i: (0, 0)),
            pl.BlockSpec((8, D), lambda i: (0, 0)),
        ],
        out_specs=pl.BlockSpec(memory_space=pl.ANY),
        out_shape=jax.ShapeDtypeStruct((N, D), jnp.float32),
        scratch_shapes=[
            pltpu.VMEM((NC, N, D), jnp.float32),
            pltpu.VMEM((N, D), jnp.float32),
        ],
    )(sp, xr, st, b1b, gb, btb)


def _dense3_body(sp_ref, w2_ref, b2_ref, o_ref, sp_v):
    pltpu.sync_copy(sp_ref, sp_v)
    hsum = sp_v[0] + sp_v[1]
    logits = jnp.dot(hsum, w2_ref[...],
                     preferred_element_type=jnp.float32) + b2_ref[0:1, :]
    m = jnp.max(logits, axis=1, keepdims=True)
    shifted = logits - m
    lse = jnp.log(jnp.sum(jnp.exp(shifted), axis=1, keepdims=True))
    o_ref[...] = shifted - lse


def _dense3(sp, W2, b2b):
    return pl.pallas_call(
        _dense3_body,
        grid=(1,),
        in_specs=[
            pl.BlockSpec(memory_space=pl.ANY),
            pl.BlockSpec((D, NCLASS), lambda i: (0, 0)),
            pl.BlockSpec((8, NCLASS), lambda i: (0, 0)),
        ],
        out_specs=pl.BlockSpec((N, NCLASS), lambda i: (0, 0)),
        out_shape=jax.ShapeDtypeStruct((N, NCLASS), jnp.float32),
        scratch_shapes=[
            pltpu.VMEM((NC, N, D), jnp.float32),
        ],
    )(sp, W2, b2b)


def kernel(x, edge_index, edge_weight, W1, b1, W2, b2, Wb, gamma, beta):
    e4 = edge_index.reshape(2, NW, NCHUNK, CH)
    w2d = edge_weight.reshape(NW, EPW)
    zeros = jnp.zeros((N, D), jnp.float32)
    b1b = jnp.broadcast_to(b1.reshape(1, D), (8, D))
    gb = jnp.broadcast_to(gamma.reshape(1, D), (8, D))
    btb = jnp.broadcast_to(beta.reshape(1, D), (8, D))
    b2b = jnp.broadcast_to(b2.reshape(1, NCLASS), (8, NCLASS))

    spmm = _get_spmm_sc()
    s1, xr, st = _dense1(x, W1, Wb)
    sp1 = spmm(s1, e4, w2d, zeros)
    h = _dense2(sp1, xr, st, b1b, gb, btb)
    sp2 = spmm(h, e4, w2d, zeros)
    return _dense3(sp2, W2, b2b)
